# Initial kernel scaffold; baseline (speedup 1.0000x reference)
import jax
import jax.numpy as jnp
from jax import lax
from jax.experimental import pallas as pl
from jax.experimental.pallas import tpu as pltpu
from jax.experimental.pallas import tpu_sc as plsc

N = 100000
NC, NS = 2, 16
CHUNK = 128
K = 8
SUP = K * CHUNK
R = 100352
ZROWS = R // NS
BN = 2000

_mesh = plsc.VectorSubcoreMesh(
    core_axis_name="c", subcore_axis_name="s", num_cores=NC, num_subcores=NS)


def _edge_loop(src2d, dst2d, tab, acc, sidx, didx, rows, semg, sems,
               row_base, n_sup):
  @pl.loop(0, n_sup)
  def _sup(j):
    r0 = row_base + j * K
    pltpu.sync_copy(src2d.at[pl.ds(r0, K)], sidx)
    pltpu.sync_copy(dst2d.at[pl.ds(r0, K)], didx)
    gs = [pltpu.async_copy(tab.at[sidx.at[k]], rows.at[k], semg)
          for k in range(K)]
    for g in gs:
      g.wait()
    ss = [pltpu.async_copy(rows.at[k], acc.at[didx.at[k]], sems, add=True)
          for k in range(K)]
    for s in ss:
      s.wait()


def _make_scatter(F, feature_split, total_rows):
  def body(src2d, dst2d, ta, tb, zer, out, sidx, didx, rows, acc, semg, sems):
    c = lax.axis_index("c")
    s = lax.axis_index("s")
    sl = pl.ds(s * ZROWS, ZROWS)
    pltpu.sync_copy(zer, acc.at[sl])
    plsc.subcore_barrier()
    if feature_split:
      rpt = total_rows // NS
      base = s * rpt
      pl.when(c == 0)(lambda: _edge_loop(
          src2d, dst2d, ta, acc, sidx, didx, rows, semg, sems, base, rpt // K))
      pl.when(c == 1)(lambda: _edge_loop(
          src2d, dst2d, tb, acc, sidx, didx, rows, semg, sems, base, rpt // K))
    else:
      rpt = total_rows // (NC * NS)
      base = (s * NC + c) * rpt
      _edge_loop(src2d, dst2d, ta, acc, sidx, didx, rows, semg, sems,
                 base, rpt // K)
    plsc.subcore_barrier()
    pl.when(c == 0)(lambda: pltpu.sync_copy(acc.at[sl], out.at[0, sl]))
    pl.when(c == 1)(lambda: pltpu.sync_copy(acc.at[sl], out.at[1, sl]))

  return pl.kernel(
      body,
      out_type=jax.ShapeDtypeStruct((NC, R, F), jnp.float32),
      mesh=_mesh,
      scratch_types=[
          pltpu.VMEM((K, CHUNK), jnp.int32),
          pltpu.VMEM((K, CHUNK), jnp.int32),
          pltpu.VMEM((K, CHUNK, F), jnp.float32),
          pltpu.VMEM_SHARED((R, F), jnp.float32),
          pltpu.SemaphoreType.DMA,
          pltpu.SemaphoreType.DMA,
      ],
  )


def _make_deg(total_rows):
  def body(dst2d, zer, out, didx, ones_v, acc, sems):
    c = lax.axis_index("c")
    s = lax.axis_index("s")
    sl = pl.ds(s * ZROWS, ZROWS)
    for i in range(CHUNK // 16):
      ones_v[pl.ds(i * 16, 16)] = jnp.ones((16,), jnp.float32)
    pltpu.sync_copy(zer, acc.at[sl])
    plsc.subcore_barrier()
    rpt = total_rows // (NC * NS)
    base = (s * NC + c) * rpt

    @pl.loop(0, rpt // K)
    def _sup(j):
      r0 = base + j * K
      pltpu.sync_copy(dst2d.at[pl.ds(r0, K)], didx)
      ss = [pltpu.async_copy(ones_v, acc.at[didx.at[k]], sems, add=True)
            for k in range(K)]
      for s_ in ss:
        s_.wait()

    plsc.subcore_barrier()
    pl.when(c == 0)(lambda: pltpu.sync_copy(acc.at[sl], out.at[0, sl]))
    pl.when(c == 1)(lambda: pltpu.sync_copy(acc.at[sl], out.at[1, sl]))

  return pl.kernel(
      body,
      out_type=jax.ShapeDtypeStruct((NC, R), jnp.float32),
      mesh=_mesh,
      scratch_types=[
          pltpu.VMEM((K, CHUNK), jnp.int32),
          pltpu.VMEM((CHUNK,), jnp.float32),
          pltpu.VMEM_SHARED((R,), jnp.float32),
          pltpu.SemaphoreType.DMA,
      ],
  )


def _mm1_body(x_ref, w_ref, o_ref):
  o_ref[...] = jnp.dot(x_ref[...], w_ref[...],
                       preferred_element_type=jnp.float32)


def _scale1_body(h_ref, dp0_ref, dp1_ref, hsa_ref, hsb_ref, dis_ref):
  dis = lax.rsqrt(dp0_ref[...] + dp1_ref[...] + 1.0)
  hs = h_ref[...] * dis
  hsa_ref[...] = hs[:, :16]
  hsb_ref[...] = hs[:, 16:]
  dis_ref[...] = dis


def _mid_body(a0_ref, a1_ref, hsa_ref, hsb_ref, dis_ref, w2_ref, b1_ref,
              hs2_ref):
  dis = dis_ref[...]
  b1 = b1_ref[...]
  r0 = jnp.maximum((a0_ref[...] + hsa_ref[...]) * dis + b1[:, :16], 0.0)
  r1 = jnp.maximum((a1_ref[...] + hsb_ref[...]) * dis + b1[:, 16:], 0.0)
  w2 = w2_ref[...]
  h2 = (jnp.dot(r0, w2[:16, :], preferred_element_type=jnp.float32)
        + jnp.dot(r1, w2[16:, :], preferred_element_type=jnp.float32))
  hs2_ref[...] = h2 * dis


def _post_body(a0_ref, a1_ref, hs2_ref, dis_ref, b2_ref, o_ref):
  o_ref[...] = ((a0_ref[...] + a1_ref[...] + hs2_ref[...]) * dis_ref[...]
                + b2_ref[...])


def _row_block(F):
  return pl.BlockSpec((BN, F), lambda i: (i, 0))


def _full_block(shape):
  return pl.BlockSpec(shape, lambda i: (0, 0))


def kernel(x, edge_index, W1, b1, W2, b2):
  x = x.astype(jnp.float32)
  ei = edge_index.astype(jnp.int32)
  E = ei.shape[1]
  group = NC * NS * SUP
  E_pad = ((E + group - 1) // group) * group
  pad = E_pad - E
  src = jnp.concatenate([ei[0], jnp.zeros((pad,), jnp.int32)])
  dst = jnp.concatenate([ei[1], jnp.full((pad,), N, jnp.int32)])
  src2d = src.reshape(-1, CHUNK)
  dst2d = dst.reshape(-1, CHUNK)
  total_rows = E_pad // CHUNK
  z16 = jnp.zeros((ZROWS, 16), jnp.float32)
  z20 = jnp.zeros((ZROWS, 20), jnp.float32)
  zflat = jnp.zeros((ZROWS,), jnp.float32)

  grid = (N // BN,)

  degp = _make_deg(total_rows)(dst2d, zflat)
  h1 = pl.pallas_call(
      _mm1_body, grid=grid,
      in_specs=[_row_block(20), _full_block((20, 32))],
      out_specs=_row_block(32),
      out_shape=jax.ShapeDtypeStruct((N, 32), jnp.float32))(x, W1)

  dp0 = degp[0, :N].reshape(N, 1)
  dp1 = degp[1, :N].reshape(N, 1)
  hsa, hsb, dis = pl.pallas_call(
      _scale1_body, grid=grid,
      in_specs=[_row_block(32), _row_block(1), _row_block(1)],
      out_specs=[_row_block(16), _row_block(16), _row_block(1)],
      out_shape=[jax.ShapeDtypeStruct((N, 16), jnp.float32),
                 jax.ShapeDtypeStruct((N, 16), jnp.float32),
                 jax.ShapeDtypeStruct((N, 1), jnp.float32)])(h1, dp0, dp1)

  agg1 = _make_scatter(16, True, total_rows)(src2d, dst2d, hsa, hsb, z16)
  a10 = agg1[0, :N]
  a11 = agg1[1, :N]

  hs2 = pl.pallas_call(
      _mid_body, grid=grid,
      in_specs=[_row_block(16), _row_block(16), _row_block(16),
                _row_block(16), _row_block(1), _full_block((32, 20)),
                _full_block((1, 32))],
      out_specs=_row_block(20),
      out_shape=jax.ShapeDtypeStruct((N, 20), jnp.float32))(
          a10, a11, hsa, hsb, dis, W2, b1.reshape(1, 32))

  agg2 = _make_scatter(20, False, total_rows)(src2d, dst2d, hs2, hs2, z20)
  a20 = agg2[0, :N]
  a21 = agg2[1, :N]

  out = pl.pallas_call(
      _post_body, grid=grid,
      in_specs=[_row_block(20), _row_block(20), _row_block(20),
                _row_block(1), _full_block((1, 20))],
      out_specs=_row_block(20),
      out_shape=jax.ShapeDtypeStruct((N, 20), jnp.float32))(
          a20, a21, hs2, dis, b2.reshape(1, 20))
  return out


# speedup vs baseline: 33.9717x; 33.9717x over previous
import jax
import jax.numpy as jnp
from jax import lax
from jax.experimental import pallas as pl
from jax.experimental.pallas import tpu as pltpu
from jax.experimental.pallas import tpu_sc as plsc

N = 100000
NC, NS = 2, 16
CHUNK = 128
K = 8
SUP = K * CHUNK
R = 100352
ZROWS = R // NS
BN = 2000

_mesh = plsc.VectorSubcoreMesh(
    core_axis_name="c", subcore_axis_name="s", num_cores=NC, num_subcores=NS)


def _edge_loop(src2d, dst2d, tab, acc, sidx, didx, rows, semg, sems,
               row_base, n_sup):
  @pl.loop(0, n_sup)
  def _sup(j):
    r0 = row_base + j * K
    pltpu.sync_copy(src2d.at[pl.ds(r0, K)], sidx)
    pltpu.sync_copy(dst2d.at[pl.ds(r0, K)], didx)
    gs = [pltpu.async_copy(tab.at[sidx.at[k]], rows.at[k], semg)
          for k in range(K)]
    for g in gs:
      g.wait()
    ss = [pltpu.async_copy(rows.at[k], acc.at[didx.at[k]], sems, add=True)
          for k in range(K)]
    for s in ss:
      s.wait()


def _make_scatter(F, total_rows):
  def body(src2d, dst2d, ta, tb, zer, out, sidx, didx, rows, acc, semg, sems):
    c = lax.axis_index("c")
    s = lax.axis_index("s")
    sl = pl.ds(s * ZROWS, ZROWS)
    pltpu.sync_copy(zer, acc.at[sl])
    plsc.subcore_barrier()
    rpt = total_rows // NS
    base = s * rpt
    pl.when(c == 0)(lambda: _edge_loop(
        src2d, dst2d, ta, acc, sidx, didx, rows, semg, sems, base, rpt // K))
    pl.when(c == 1)(lambda: _edge_loop(
        src2d, dst2d, tb, acc, sidx, didx, rows, semg, sems, base, rpt // K))
    plsc.subcore_barrier()
    pl.when(c == 0)(lambda: pltpu.sync_copy(acc.at[sl], out.at[0, sl]))
    pl.when(c == 1)(lambda: pltpu.sync_copy(acc.at[sl], out.at[1, sl]))

  return pl.kernel(
      body,
      out_type=jax.ShapeDtypeStruct((NC, R, F), jnp.float32),
      mesh=_mesh,
      compiler_params=pltpu.CompilerParams(use_tc_tiling_on_sc=False),
      scratch_types=[
          pltpu.VMEM((K, CHUNK), jnp.int32),
          pltpu.VMEM((K, CHUNK), jnp.int32),
          pltpu.VMEM((K, CHUNK, F), jnp.float32),
          pltpu.VMEM_SHARED((R, F), jnp.float32),
          pltpu.SemaphoreType.DMA,
          pltpu.SemaphoreType.DMA,
      ],
  )


def _make_deg(total_rows):
  def body(dst2d, zer, out, didx, ones_v, acc, sems):
    c = lax.axis_index("c")
    s = lax.axis_index("s")
    sl = pl.ds(s * ZROWS, ZROWS)
    for i in range(CHUNK // 16):
      ones_v[pl.ds(i * 16, 16)] = jnp.ones((16,), jnp.float32)
    pltpu.sync_copy(zer, acc.at[sl])
    plsc.subcore_barrier()
    rpt = total_rows // (NC * NS)
    base = (s * NC + c) * rpt

    @pl.loop(0, rpt // K)
    def _sup(j):
      r0 = base + j * K
      pltpu.sync_copy(dst2d.at[pl.ds(r0, K)], didx)
      ss = [pltpu.async_copy(ones_v, acc.at[didx.at[k]], sems, add=True)
            for k in range(K)]
      for s_ in ss:
        s_.wait()

    plsc.subcore_barrier()
    pl.when(c == 0)(lambda: pltpu.sync_copy(acc.at[sl], out.at[0, sl]))
    pl.when(c == 1)(lambda: pltpu.sync_copy(acc.at[sl], out.at[1, sl]))

  return pl.kernel(
      body,
      out_type=jax.ShapeDtypeStruct((NC, R), jnp.float32),
      mesh=_mesh,
      compiler_params=pltpu.CompilerParams(use_tc_tiling_on_sc=False),
      scratch_types=[
          pltpu.VMEM((K, CHUNK), jnp.int32),
          pltpu.VMEM((CHUNK,), jnp.float32),
          pltpu.VMEM_SHARED((R,), jnp.float32),
          pltpu.SemaphoreType.DMA,
      ],
  )


def _mm1_body(x_ref, w_ref, o_ref):
  o_ref[...] = jnp.dot(x_ref[...], w_ref[...],
                       preferred_element_type=jnp.float32)


def _scale1_body(h_ref, dp0_ref, dp1_ref, hsa_ref, hsb_ref, dis_ref):
  dis = lax.rsqrt(dp0_ref[...] + dp1_ref[...] + 1.0)
  hs = h_ref[...] * dis
  hsa_ref[...] = hs[:, :16]
  hsb_ref[...] = hs[:, 16:]
  dis_ref[...] = dis


def _mid_body(a0_ref, a1_ref, hsa_ref, hsb_ref, dis_ref, w2_ref, b1_ref,
              hs2a_ref, hs2b_ref):
  dis = dis_ref[...]
  b1 = b1_ref[...]
  r0 = jnp.maximum((a0_ref[...] + hsa_ref[...]) * dis + b1[:, :16], 0.0)
  r1 = jnp.maximum((a1_ref[...] + hsb_ref[...]) * dis + b1[:, 16:], 0.0)
  w2 = w2_ref[...]
  h2 = (jnp.dot(r0, w2[:16, :], preferred_element_type=jnp.float32)
        + jnp.dot(r1, w2[16:, :], preferred_element_type=jnp.float32))
  hs2 = h2 * dis
  zpad = jnp.zeros((hs2.shape[0], 6), jnp.float32)
  hs2a_ref[...] = jnp.concatenate([hs2[:, :10], zpad], axis=-1)
  hs2b_ref[...] = jnp.concatenate([hs2[:, 10:], zpad], axis=-1)


def _post_body(a0_ref, a1_ref, hs2a_ref, hs2b_ref, dis_ref, b2_ref, o_ref):
  dis = dis_ref[...]
  b2 = b2_ref[...]
  v0 = (a0_ref[...] + hs2a_ref[...])[:, :10] * dis + b2[:, :10]
  v1 = (a1_ref[...] + hs2b_ref[...])[:, :10] * dis + b2[:, 10:]
  o_ref[...] = jnp.concatenate([v0, v1], axis=-1)


def _row_block(F):
  return pl.BlockSpec((BN, F), lambda i: (i, 0))


def _full_block(shape):
  return pl.BlockSpec(shape, lambda i: (0, 0))


def kernel(x, edge_index, W1, b1, W2, b2):
  x = x.astype(jnp.float32)
  ei = edge_index.astype(jnp.int32)
  E = ei.shape[1]
  group = NC * NS * SUP
  E_pad = ((E + group - 1) // group) * group
  pad = E_pad - E
  src = jnp.concatenate([ei[0], jnp.zeros((pad,), jnp.int32)])
  dst = jnp.concatenate([ei[1], jnp.full((pad,), N, jnp.int32)])
  src2d = src.reshape(-1, CHUNK)
  dst2d = dst.reshape(-1, CHUNK)
  total_rows = E_pad // CHUNK
  z16 = jnp.zeros((ZROWS, 16), jnp.float32)
  zflat = jnp.zeros((ZROWS,), jnp.float32)

  grid = (N // BN,)

  degp = _make_deg(total_rows)(dst2d, zflat)
  h1 = pl.pallas_call(
      _mm1_body, grid=grid,
      in_specs=[_row_block(20), _full_block((20, 32))],
      out_specs=_row_block(32),
      out_shape=jax.ShapeDtypeStruct((N, 32), jnp.float32))(x, W1)

  dp0 = degp[0, :N].reshape(N, 1)
  dp1 = degp[1, :N].reshape(N, 1)
  hsa, hsb, dis = pl.pallas_call(
      _scale1_body, grid=grid,
      in_specs=[_row_block(32), _row_block(1), _row_block(1)],
      out_specs=[_row_block(16), _row_block(16), _row_block(1)],
      out_shape=[jax.ShapeDtypeStruct((N, 16), jnp.float32),
                 jax.ShapeDtypeStruct((N, 16), jnp.float32),
                 jax.ShapeDtypeStruct((N, 1), jnp.float32)])(h1, dp0, dp1)

  agg1 = _make_scatter(16, total_rows)(src2d, dst2d, hsa, hsb, z16)
  a10 = agg1[0, :N]
  a11 = agg1[1, :N]

  hs2a, hs2b = pl.pallas_call(
      _mid_body, grid=grid,
      in_specs=[_row_block(16), _row_block(16), _row_block(16),
                _row_block(16), _row_block(1), _full_block((32, 20)),
                _full_block((1, 32))],
      out_specs=[_row_block(16), _row_block(16)],
      out_shape=[jax.ShapeDtypeStruct((N, 16), jnp.float32),
                 jax.ShapeDtypeStruct((N, 16), jnp.float32)])(
          a10, a11, hsa, hsb, dis, W2, b1.reshape(1, 32))

  agg2 = _make_scatter(16, total_rows)(src2d, dst2d, hs2a, hs2b, z16)
  a20 = agg2[0, :N]
  a21 = agg2[1, :N]

  out = pl.pallas_call(
      _post_body, grid=grid,
      in_specs=[_row_block(16), _row_block(16), _row_block(16),
                _row_block(16), _row_block(1), _full_block((1, 20))],
      out_specs=_row_block(20),
      out_shape=jax.ShapeDtypeStruct((N, 20), jnp.float32))(
          a20, a21, hs2a, hs2b, dis, b2.reshape(1, 20))
  return out


# speedup vs baseline: 39.1261x; 1.1517x over previous
import jax
import jax.numpy as jnp
from jax import lax
from jax.experimental import pallas as pl
from jax.experimental.pallas import tpu as pltpu
from jax.experimental.pallas import tpu_sc as plsc

N = 100000
NC, NS = 2, 16
CHUNK = 128
K = 4
SUP = K * CHUNK
R = 100352
ZROWS = R // NS
BN = 2000

_mesh = plsc.VectorSubcoreMesh(
    core_axis_name="c", subcore_axis_name="s", num_cores=NC, num_subcores=NS)


D = 2


def _edge_loop(src2d, dst2d, tab, acc, sidx, didx, rows, semi, semg, sems,
               row_base, n_sup):
  @pl.loop(0, n_sup // D)
  def _outer(i):
    g0 = row_base + i * (D * K)
    il = []
    for b in range(D):
      r0 = g0 + b * K
      il.append((pltpu.async_copy(src2d.at[pl.ds(r0, K)], sidx.at[b], semi),
                 pltpu.async_copy(dst2d.at[pl.ds(r0, K)], didx.at[b], semi)))
    gl = []
    for b in range(D):
      il[b][0].wait()
      il[b][1].wait()
      gl.append([pltpu.async_copy(tab.at[sidx.at[b, k]], rows.at[b, k], semg)
                 for k in range(K)])
    sl = []
    for b in range(D):
      for g in gl[b]:
        g.wait()
      sl.append([pltpu.async_copy(rows.at[b, k], acc.at[didx.at[b, k]],
                                  sems, add=True) for k in range(K)])
    for b in range(D):
      for s in sl[b]:
        s.wait()


def _make_scatter(F, total_rows):
  def body(src2d, dst2d, ta, tb, zer, out, sidx, didx, rows, acc,
           semi, semg, sems):
    c = lax.axis_index("c")
    s = lax.axis_index("s")
    sl = pl.ds(s * ZROWS, ZROWS)
    pltpu.sync_copy(zer, acc.at[sl])
    plsc.subcore_barrier()
    rpt = total_rows // NS
    base = s * rpt
    pl.when(c == 0)(lambda: _edge_loop(
        src2d, dst2d, ta, acc, sidx, didx, rows, semi, semg, sems,
        base, rpt // K))
    pl.when(c == 1)(lambda: _edge_loop(
        src2d, dst2d, tb, acc, sidx, didx, rows, semi, semg, sems,
        base, rpt // K))
    plsc.subcore_barrier()
    pl.when(c == 0)(lambda: pltpu.sync_copy(acc.at[sl], out.at[0, sl]))
    pl.when(c == 1)(lambda: pltpu.sync_copy(acc.at[sl], out.at[1, sl]))

  return pl.kernel(
      body,
      out_type=jax.ShapeDtypeStruct((NC, R, F), jnp.float32),
      mesh=_mesh,
      compiler_params=pltpu.CompilerParams(use_tc_tiling_on_sc=False),
      scratch_types=[
          pltpu.VMEM((D, K, CHUNK), jnp.int32),
          pltpu.VMEM((D, K, CHUNK), jnp.int32),
          pltpu.VMEM((D, K, CHUNK, F), jnp.float32),
          pltpu.VMEM_SHARED((R, F), jnp.float32),
          pltpu.SemaphoreType.DMA,
          pltpu.SemaphoreType.DMA,
          pltpu.SemaphoreType.DMA,
      ],
  )


def _make_deg(total_rows):
  def body(dst2d, zer, out, didx, ones_v, acc, semi, sems):
    c = lax.axis_index("c")
    s = lax.axis_index("s")
    sl = pl.ds(s * ZROWS, ZROWS)
    for i in range(CHUNK // 16):
      ones_v[pl.ds(i * 16, 16)] = jnp.ones((16,), jnp.float32)
    pltpu.sync_copy(zer, acc.at[sl])
    plsc.subcore_barrier()
    rpt = total_rows // (NC * NS)
    base = (s * NC + c) * rpt
    DD = 2

    @pl.loop(0, rpt // (K * DD))
    def _sup(i):
      g0 = base + i * (DD * K)
      il = [pltpu.async_copy(dst2d.at[pl.ds(g0 + b * K, K)], didx.at[b], semi)
            for b in range(DD)]
      sl_ = []
      for b in range(DD):
        il[b].wait()
        sl_.append([pltpu.async_copy(ones_v, acc.at[didx.at[b, k]],
                                     sems, add=True) for k in range(K)])
      for b in range(DD):
        for s_ in sl_[b]:
          s_.wait()

    plsc.subcore_barrier()
    pl.when(c == 0)(lambda: pltpu.sync_copy(acc.at[sl], out.at[0, sl]))
    pl.when(c == 1)(lambda: pltpu.sync_copy(acc.at[sl], out.at[1, sl]))

  return pl.kernel(
      body,
      out_type=jax.ShapeDtypeStruct((NC, R), jnp.float32),
      mesh=_mesh,
      compiler_params=pltpu.CompilerParams(use_tc_tiling_on_sc=False),
      scratch_types=[
          pltpu.VMEM((2, K, CHUNK), jnp.int32),
          pltpu.VMEM((CHUNK,), jnp.float32),
          pltpu.VMEM_SHARED((R,), jnp.float32),
          pltpu.SemaphoreType.DMA,
          pltpu.SemaphoreType.DMA,
      ],
  )


def _mm1_body(x_ref, w_ref, o_ref):
  o_ref[...] = jnp.dot(x_ref[...], w_ref[...],
                       preferred_element_type=jnp.float32)


def _scale1_body(h_ref, dp0_ref, dp1_ref, hsa_ref, hsb_ref, dis_ref):
  dis = lax.rsqrt(dp0_ref[...] + dp1_ref[...] + 1.0)
  hs = h_ref[...] * dis
  hsa_ref[...] = hs[:, :16]
  hsb_ref[...] = hs[:, 16:]
  dis_ref[...] = dis


def _mid_body(a0_ref, a1_ref, hsa_ref, hsb_ref, dis_ref, w2_ref, b1_ref,
              hs2a_ref, hs2b_ref):
  dis = dis_ref[...]
  b1 = b1_ref[...]
  r0 = jnp.maximum((a0_ref[...] + hsa_ref[...]) * dis + b1[:, :16], 0.0)
  r1 = jnp.maximum((a1_ref[...] + hsb_ref[...]) * dis + b1[:, 16:], 0.0)
  w2 = w2_ref[...]
  h2 = (jnp.dot(r0, w2[:16, :], preferred_element_type=jnp.float32)
        + jnp.dot(r1, w2[16:, :], preferred_element_type=jnp.float32))
  hs2 = h2 * dis
  zpad = jnp.zeros((hs2.shape[0], 6), jnp.float32)
  hs2a_ref[...] = jnp.concatenate([hs2[:, :10], zpad], axis=-1)
  hs2b_ref[...] = jnp.concatenate([hs2[:, 10:], zpad], axis=-1)


def _post_body(a0_ref, a1_ref, hs2a_ref, hs2b_ref, dis_ref, b2_ref, o_ref):
  dis = dis_ref[...]
  b2 = b2_ref[...]
  v0 = (a0_ref[...] + hs2a_ref[...])[:, :10] * dis + b2[:, :10]
  v1 = (a1_ref[...] + hs2b_ref[...])[:, :10] * dis + b2[:, 10:]
  o_ref[...] = jnp.concatenate([v0, v1], axis=-1)


def _row_block(F):
  return pl.BlockSpec((BN, F), lambda i: (i, 0))


def _full_block(shape):
  return pl.BlockSpec(shape, lambda i: (0, 0))


def kernel(x, edge_index, W1, b1, W2, b2):
  x = x.astype(jnp.float32)
  ei = edge_index.astype(jnp.int32)
  E = ei.shape[1]
  group = NC * NS * SUP
  E_pad = ((E + group - 1) // group) * group
  pad = E_pad - E
  src = jnp.concatenate([ei[0], jnp.zeros((pad,), jnp.int32)])
  dst = jnp.concatenate([ei[1], jnp.full((pad,), N, jnp.int32)])
  src2d = src.reshape(-1, CHUNK)
  dst2d = dst.reshape(-1, CHUNK)
  total_rows = E_pad // CHUNK
  z16 = jnp.zeros((ZROWS, 16), jnp.float32)
  zflat = jnp.zeros((ZROWS,), jnp.float32)

  grid = (N // BN,)

  degp = _make_deg(total_rows)(dst2d, zflat)
  h1 = pl.pallas_call(
      _mm1_body, grid=grid,
      in_specs=[_row_block(20), _full_block((20, 32))],
      out_specs=_row_block(32),
      out_shape=jax.ShapeDtypeStruct((N, 32), jnp.float32))(x, W1)

  dp0 = degp[0, :N].reshape(N, 1)
  dp1 = degp[1, :N].reshape(N, 1)
  hsa, hsb, dis = pl.pallas_call(
      _scale1_body, grid=grid,
      in_specs=[_row_block(32), _row_block(1), _row_block(1)],
      out_specs=[_row_block(16), _row_block(16), _row_block(1)],
      out_shape=[jax.ShapeDtypeStruct((N, 16), jnp.float32),
                 jax.ShapeDtypeStruct((N, 16), jnp.float32),
                 jax.ShapeDtypeStruct((N, 1), jnp.float32)])(h1, dp0, dp1)

  agg1 = _make_scatter(16, total_rows)(src2d, dst2d, hsa, hsb, z16)
  a10 = agg1[0, :N]
  a11 = agg1[1, :N]

  hs2a, hs2b = pl.pallas_call(
      _mid_body, grid=grid,
      in_specs=[_row_block(16), _row_block(16), _row_block(16),
                _row_block(16), _row_block(1), _full_block((32, 20)),
                _full_block((1, 32))],
      out_specs=[_row_block(16), _row_block(16)],
      out_shape=[jax.ShapeDtypeStruct((N, 16), jnp.float32),
                 jax.ShapeDtypeStruct((N, 16), jnp.float32)])(
          a10, a11, hsa, hsb, dis, W2, b1.reshape(1, 32))

  agg2 = _make_scatter(16, total_rows)(src2d, dst2d, hs2a, hs2b, z16)
  a20 = agg2[0, :N]
  a21 = agg2[1, :N]

  out = pl.pallas_call(
      _post_body, grid=grid,
      in_specs=[_row_block(16), _row_block(16), _row_block(16),
                _row_block(16), _row_block(1), _full_block((1, 20))],
      out_specs=_row_block(20),
      out_shape=jax.ShapeDtypeStruct((N, 20), jnp.float32))(
          a20, a21, hs2a, hs2b, dis, b2.reshape(1, 20))
  return out
